# baseline (speedup 1.0000x reference)
import functools
import math

import jax
import jax.numpy as jnp
from jax import lax
from jax.experimental import pallas as pl
from jax.experimental.pallas import tpu as pltpu
from jax.experimental.pallas import tpu_sc as plsc

API_DIM = 32
CLASS_DIM = 32
FINAL_DIM = API_DIM + CLASS_DIM
SCALE = math.sqrt(FINAL_DIM)

NC = 2
NS = 16
NW = NC * NS
CHUNK = 128
GROUP = 128 // API_DIM


def _sc_embed(n_rows, hist):
    n_chunks = n_rows // (NW * CHUNK)
    rows_per_w = n_chunks * CHUNK
    mesh = plsc.VectorSubcoreMesh(core_axis_name="c", subcore_axis_name="s")

    @functools.partial(
        pl.kernel,
        out_type=jax.ShapeDtypeStruct((n_rows, FINAL_DIM), jnp.float32),
        mesh=mesh,
        scratch_types=[
            pltpu.VMEM((n_chunks, CHUNK), jnp.int32),
            pltpu.VMEM((n_chunks, CHUNK), jnp.int32),
            pltpu.VMEM((CHUNK, 128), jnp.float32),
            pltpu.VMEM((CHUNK, 128), jnp.float32),
            pltpu.VMEM((CHUNK, FINAL_DIM), jnp.float32),
            pltpu.VMEM((CHUNK, 128), jnp.float32),
            pltpu.VMEM((CHUNK, 128), jnp.float32),
            pltpu.VMEM((CHUNK, FINAL_DIM), jnp.float32),
            pltpu.VMEM((CHUNK,), jnp.int32),
            pltpu.VMEM((CHUNK,), jnp.int32),
            pltpu.VMEM((CHUNK,), jnp.int32),
            pltpu.VMEM((CHUNK,), jnp.int32),
            pltpu.VMEM((CHUNK,), jnp.int32),
            pltpu.VMEM((CHUNK,), jnp.int32),
            pltpu.VMEM((CHUNK,), jnp.int32),
            pltpu.VMEM((CHUNK,), jnp.int32),
            pltpu.SemaphoreType.DMA,
            pltpu.SemaphoreType.DMA,
            pltpu.SemaphoreType.DMA,
            pltpu.SemaphoreType.DMA,
        ],
        compiler_params=pltpu.CompilerParams(
            use_tc_tiling_on_sc=True, needs_layout_passes=False),
    )
    def k(cls_idx, api_idx, cls_tab, api_tab, out,
          idxc_c, idxc_a, g_c0, g_a0, out_v0, g_c1, g_a1, out_v1,
          idxg_c0, idxg_a0, subc0, suba0, idxg_c1, idxg_a1, subc1, suba1,
          sem_g0, sem_g1, sem_w0, sem_w1):
        wid = lax.axis_index("s") * NC + lax.axis_index("c")
        rbase = wid * rows_per_w
        pltpu.sync_copy(cls_idx.at[wid], idxc_c)
        pltpu.sync_copy(api_idx.at[wid], idxc_a)

        bufs = ((g_c0, g_a0, out_v0, idxg_c0, idxg_a0, subc0, suba0,
                 sem_g0, sem_w0),
                (g_c1, g_a1, out_v1, idxg_c1, idxg_a1, subc1, suba1,
                 sem_g1, sem_w1))

        def gather_start(j, p):
            g_c, g_a, _, idxg_c, idxg_a, subc, suba, sem_g, _ = bufs[p]
            for v8 in range(CHUNK // 16):
                sl = pl.ds(v8 * 16, 16)
                ic = idxc_c[j, sl]
                ia = idxc_a[j, sl]
                idxg_c[sl] = lax.shift_right_logical(ic, 2)
                idxg_a[sl] = lax.shift_right_logical(ia, 2)
                subc[sl] = lax.shift_left(lax.bitwise_and(ic, 3), 5)
                suba[sl] = lax.shift_left(lax.bitwise_and(ia, 3), 5)
            pltpu.async_copy(cls_tab.at[idxg_c], g_c, sem_g)
            pltpu.async_copy(api_tab.at[idxg_a], g_a, sem_g)

        gather_start(0, 0)
        iota16 = jax.lax.broadcasted_iota(jnp.int32, (16,), 0)

        def pair_body(jj, carry):
            for p in range(2):
                j = jj * 2 + p
                (g_c, g_a, out_v, idxg_c, idxg_a, subc, suba,
                 sem_g, sem_w) = bufs[p]

                @pl.when(j + 1 < n_chunks)
                def _():
                    gather_start(j + 1, 1 - p)

                pltpu.make_async_copy(cls_tab.at[idxg_c], g_c, sem_g).wait()
                pltpu.make_async_copy(api_tab.at[idxg_a], g_a, sem_g).wait()

                @pl.when(j >= 2)
                def _():
                    pltpu.make_async_copy(
                        out_v, out.at[pl.ds(rbase, CHUNK)], sem_w).wait()

                for i in range(CHUNK):
                    fi = jnp.full((16,), i, jnp.int32)
                    c0 = plsc.load_gather(subc, [fi]) + iota16
                    a0 = plsc.load_gather(suba, [fi]) + iota16
                    out_v[i, pl.ds(0, 16)] = (
                        plsc.load_gather(g_c, [fi, c0]) * SCALE)
                    out_v[i, pl.ds(16, 16)] = (
                        plsc.load_gather(g_c, [fi, c0 + 16]) * SCALE)
                    out_v[i, pl.ds(32, 16)] = (
                        plsc.load_gather(g_a, [fi, a0]) * SCALE)
                    out_v[i, pl.ds(48, 16)] = (
                        plsc.load_gather(g_a, [fi, a0 + 16]) * SCALE)

                pltpu.async_copy(
                    out_v, out.at[pl.ds(rbase + j * CHUNK, CHUNK)], sem_w)
            return carry

        lax.fori_loop(0, n_chunks // 2, pair_body, 0)
        for p in range(2):
            out_v, sem_w = bufs[p][2], bufs[p][8]
            pltpu.make_async_copy(
                out_v, out.at[pl.ds(rbase, CHUNK)], sem_w).wait()

    return k


def kernel(class_ids, api_ids, class_table, api_table):
    batch, hist = class_ids.shape
    n_rows = batch * hist
    assert n_rows % (NW * CHUNK) == 0
    n_chunks = n_rows // (NW * CHUNK)

    def prep_ids(ids):
        return ids.astype(jnp.int32).reshape(NW, n_chunks, CHUNK)

    rm_cls = class_table.reshape(class_table.shape[0] // GROUP, 128)
    rm_api = api_table.reshape(api_table.shape[0] // GROUP, 128)
    out = _sc_embed(n_rows, hist)(
        prep_ids(class_ids), prep_ids(api_ids), rm_cls, rm_api)
    return out.reshape(batch, hist, FINAL_DIM)


# speedup vs baseline: 1.4778x; 1.4778x over previous
import functools
import math

import jax
import jax.numpy as jnp
from jax import lax
from jax.experimental import pallas as pl
from jax.experimental.pallas import tpu as pltpu
from jax.experimental.pallas import tpu_sc as plsc

API_DIM = 32
CLASS_DIM = 32
FINAL_DIM = API_DIM + CLASS_DIM
SCALE = math.sqrt(FINAL_DIM)

NC = 2
NS = 16
NW = NC * NS
CHUNK = 128


def _sc_embed(n_rows, n_chunks_per_w):
    rows_per_w = n_chunks_per_w * CHUNK
    mesh = plsc.VectorSubcoreMesh(core_axis_name="c", subcore_axis_name="s")

    @functools.partial(
        pl.kernel,
        out_type=jax.ShapeDtypeStruct((n_rows, FINAL_DIM), jnp.float32),
        mesh=mesh,
        scratch_types=[
            pltpu.VMEM((n_chunks_per_w, CHUNK), jnp.int32),
            pltpu.VMEM((n_chunks_per_w, CHUNK), jnp.int32),
            pltpu.VMEM((CHUNK, CLASS_DIM), jnp.float32),
            pltpu.VMEM((CHUNK, API_DIM), jnp.float32),
            pltpu.VMEM((CHUNK, FINAL_DIM), jnp.float32),
            pltpu.VMEM((CHUNK, CLASS_DIM), jnp.float32),
            pltpu.VMEM((CHUNK, API_DIM), jnp.float32),
            pltpu.VMEM((CHUNK, FINAL_DIM), jnp.float32),
            pltpu.SemaphoreType.DMA,
            pltpu.SemaphoreType.DMA,
            pltpu.SemaphoreType.DMA,
            pltpu.SemaphoreType.DMA,
        ],
        compiler_params=pltpu.CompilerParams(use_tc_tiling_on_sc=False),
    )
    def k(cls_ids, api_ids, cls_tab, api_tab, out,
          idx_cls, idx_api, cls_v0, api_v0, out_v0, cls_v1, api_v1, out_v1,
          sem_g0, sem_g1, sem_w0, sem_w1):
        wid = lax.axis_index("s") * NC + lax.axis_index("c")
        cbase = wid * n_chunks_per_w
        rbase = wid * rows_per_w
        pltpu.sync_copy(cls_ids.at[pl.ds(cbase, n_chunks_per_w)], idx_cls)
        pltpu.sync_copy(api_ids.at[pl.ds(cbase, n_chunks_per_w)], idx_api)

        bufs = ((cls_v0, api_v0, out_v0, sem_g0, sem_w0),
                (cls_v1, api_v1, out_v1, sem_g1, sem_w1))

        def gather_start(j, p):
            cls_v, api_v, _, sem_g, _ = bufs[p]
            pltpu.async_copy(cls_tab.at[idx_cls.at[j]], cls_v, sem_g)
            pltpu.async_copy(api_tab.at[idx_api.at[j]], api_v, sem_g)

        gather_start(0, 0)

        def pair_body(jj, carry):
            for p in range(2):
                j = jj * 2 + p
                cls_v, api_v, out_v, sem_g, sem_w = bufs[p]

                @pl.when(j + 1 < n_chunks_per_w)
                def _():
                    gather_start(j + 1, 1 - p)

                pltpu.make_async_copy(
                    cls_tab.at[idx_cls.at[j]], cls_v, sem_g).wait()
                pltpu.make_async_copy(
                    api_tab.at[idx_api.at[j]], api_v, sem_g).wait()

                @pl.when(j >= 2)
                def _():
                    pltpu.make_async_copy(
                        out_v, out.at[pl.ds(rbase, CHUNK)], sem_w).wait()

                for i in range(CHUNK):
                    out_v[i, pl.ds(0, 16)] = cls_v[i, pl.ds(0, 16)] * SCALE
                    out_v[i, pl.ds(16, 16)] = cls_v[i, pl.ds(16, 16)] * SCALE
                    out_v[i, pl.ds(32, 16)] = api_v[i, pl.ds(0, 16)] * SCALE
                    out_v[i, pl.ds(48, 16)] = api_v[i, pl.ds(16, 16)] * SCALE

                pltpu.async_copy(
                    out_v, out.at[pl.ds(rbase + j * CHUNK, CHUNK)], sem_w)
            return carry

        lax.fori_loop(0, n_chunks_per_w // 2, pair_body, 0)
        for p in range(2):
            _, _, out_v, _, sem_w = bufs[p]
            pltpu.make_async_copy(
                out_v, out.at[pl.ds(rbase, CHUNK)], sem_w).wait()

    return k


def kernel(class_ids, api_ids, class_table, api_table):
    batch, hist = class_ids.shape
    n_rows = batch * hist
    assert n_rows % (NW * CHUNK) == 0
    n_chunks_per_w = n_rows // (NW * CHUNK)
    cls_flat = class_ids.reshape(n_rows // CHUNK, CHUNK).astype(jnp.int32)
    api_flat = api_ids.reshape(n_rows // CHUNK, CHUNK).astype(jnp.int32)
    out = _sc_embed(n_rows, n_chunks_per_w)(
        cls_flat, api_flat, class_table, api_table)
    return out.reshape(batch, hist, FINAL_DIM)
